# Initial kernel scaffold; baseline (speedup 1.0000x reference)
import jax
import jax.numpy as jnp
from jax.experimental import pallas as pl


def kernel(queries, keys, values):
    raise NotImplementedError("write your pallas kernel here")


# speedup vs baseline: 8.9643x; 8.9643x over previous
import functools
import math

import jax
import jax.numpy as jnp
from jax import lax
from jax.experimental import pallas as pl
from jax.experimental.pallas import tpu as pltpu
from jax.experimental.pallas import tpu_sc as plsc

S = 2048
H = 12
D = 64
C = H * D
F = S // 2 + 1
FPAD = 1032
NC = 2
NS = 16
NW = NC * NS
ROWS = H * S
RPW = ROWS // NW
RB = 64


_LOSCALE = 256.0


def _bsplit(x):
    hi = x.astype(jnp.bfloat16)
    lo = ((x - hi.astype(jnp.float32)) * _LOSCALE).astype(jnp.bfloat16)
    return hi, lo


def _np_bsplit(x64):
    import numpy as np
    xf = np.asarray(x64, np.float32)
    hi = xf.astype(jnp.bfloat16)
    lo = ((xf - hi.astype(np.float32)) * np.float32(_LOSCALE)).astype(jnp.bfloat16)
    return hi, lo


def _dft_constants():
    import numpy as np
    sv = np.arange(S, dtype=np.int64)[:, None]
    fv = np.arange(FPAD, dtype=np.int64)[None, :]
    m = (sv * fv) % S
    ang = m.astype(np.float64) * (2.0 * math.pi / S)
    valid = (fv < F).astype(np.float64)
    cm = np.cos(ang) * valid
    sm = np.sin(ang) * valid
    w = np.where(fv == 0, 1.0 / S, 2.0 / S)
    w = (np.where(fv == F - 1, 1.0 / S, w) * valid).astype(np.float32)
    ch, cl = _np_bsplit(cm)
    sh, sl = _np_bsplit(sm)
    return (jnp.asarray(ch), jnp.asarray(cl), jnp.asarray(sh),
            jnp.asarray(sl), jnp.asarray(w.reshape(FPAD, 1)))


_DNT = (((0,), (0,)), ((), ()))
_DNN = (((1,), (0,)), ((), ()))


def _dot3(mh, ml, x, dn):
    xh, xl = _bsplit(x)
    d = lambda a, b: lax.dot_general(a, b, dn,
                                     preferred_element_type=jnp.float32)
    return d(mh, xh) + (d(mh, xl) + d(ml, xh)) * jnp.float32(1.0 / _LOSCALE)


def _fwd_dft_body(ch, cl, sh, sl, q_ref, k_ref, qr_ref, qi_ref, kr_ref, ki_ref):
    q = q_ref[...]
    k = k_ref[...]
    chv, clv, shv, slv = ch[...], cl[...], sh[...], sl[...]
    qr_ref[...] = _dot3(chv, clv, q, _DNT)
    qi_ref[...] = -_dot3(shv, slv, q, _DNT)
    kr_ref[...] = _dot3(chv, clv, k, _DNT)
    ki_ref[...] = -_dot3(shv, slv, k, _DNT)


def _inv_dft_body(ch, cl, sh, sl, w, qr_ref, qi_ref, kr_ref, ki_ref, corr_ref):
    qr, qi = qr_ref[...], qi_ref[...]
    kr, ki = kr_ref[...], ki_ref[...]
    wv = w[...]
    pr = (qr * kr + qi * ki) * wv
    pi = (qi * kr - qr * ki) * wv
    corr_ref[...] = (
        _dot3(ch[...], cl[...], pr, _DNN) - _dot3(sh[...], sl[...], pi, _DNN)
    )


SORT_CB = 128


def _sort_softmax_body(corr_ref, a_ref, gidx_ref, key_ref, idx_ref):
    cb = SORT_CB
    key_ref[...] = corr_ref[...]
    t2 = lax.broadcasted_iota(jnp.int32, (S, cb), 0)
    idx_ref[...] = t2

    def stage(d, ksz):
        key = key_ref[...]
        idx = idx_ref[...]
        tbit = (t2 & d) != 0
        desc = (t2 & ksz) == 0
        k_up = pltpu.roll(key, S - d, axis=0)
        k_dn = pltpu.roll(key, d, axis=0)
        i_up = pltpu.roll(idx, S - d, axis=0)
        i_dn = pltpu.roll(idx, d, axis=0)
        k_p = jnp.where(tbit, k_dn, k_up)
        i_p = jnp.where(tbit, i_dn, i_up)
        k_lo = jnp.where(tbit, k_p, key)
        k_hi = jnp.where(tbit, key, k_p)
        swap = jnp.equal(k_lo < k_hi, desc)
        key_ref[...] = jnp.where(swap, k_p, key)
        idx_ref[...] = jnp.where(swap, i_p, idx)

    def outer(k, carry):
        ksz = jnp.int32(1) << k

        def inner(j, c2):
            stage(jnp.int32(1) << (k - 1 - j), ksz)
            return c2

        lax.fori_loop(0, k, inner, 0)
        return carry

    lax.fori_loop(1, 12, outer, 0)
    key = key_ref[...]
    e = jnp.exp(key - key[0:1, :])
    a_ref[...] = e / jnp.sum(e, axis=0, keepdims=True)
    chan = pl.program_id(0) * cb + lax.broadcasted_iota(jnp.int32, (S, cb), 1)
    gidx_ref[...] = idx_ref[...] + (chan // D) * S


def _sc_gather_body(v_hbm, w_hbm, i_hbm, out_hbm, idxb, wb, rows, accv, sem):
    wid = lax.axis_index("s") * NC + lax.axis_index("c")
    base = wid * RPW

    def blk(bi, carry):
        r0 = base + bi * RB
        pltpu.sync_copy(i_hbm.at[pl.ds(r0, RB)], idxb)
        pltpu.sync_copy(w_hbm.at[pl.ds(r0, RB)], wb)

        def row(rr, c2):
            pltpu.async_copy(v_hbm.at[idxb.at[rr]], rows, sem).wait()
            wvecs = [wb[rr, pl.ds(g * 16, 16)] for g in range(4)]
            accs = [jnp.zeros((16,), jnp.float32) for _ in range(4)]
            for j in range(D):
                wj = wvecs[j // 16][j % 16]
                for c in range(4):
                    accs[c] = accs[c] + wj * rows[j, pl.ds(c * 16, 16)]
            for c in range(4):
                accv[pl.ds(c * 16, 16)] = accs[c]
            pltpu.sync_copy(accv, out_hbm.at[r0 + rr])
            return c2

        lax.fori_loop(0, RB, row, 0)
        return carry

    lax.fori_loop(0, RPW // RB, blk, 0)


def _sc_gather(v_flat, w_rows, i_rows):
    mesh = plsc.VectorSubcoreMesh(core_axis_name="c", subcore_axis_name="s")
    run = pl.kernel(
        _sc_gather_body,
        out_type=jax.ShapeDtypeStruct((ROWS, D), jnp.float32),
        mesh=mesh,
        compiler_params=pltpu.CompilerParams(use_tc_tiling_on_sc=False),
        scratch_types=[
            pltpu.VMEM((RB, D), jnp.int32),
            pltpu.VMEM((RB, D), jnp.float32),
            pltpu.VMEM((D, D), jnp.float32),
            pltpu.VMEM((D,), jnp.float32),
            pltpu.SemaphoreType.DMA,
        ],
    )
    return run(v_flat, w_rows, i_rows)


def kernel(queries, keys, values):
    q2 = jnp.moveaxis(queries[0], 0, 1).reshape(S, C)
    k2 = jnp.moveaxis(keys[0], 0, 1).reshape(S, C)
    ch, cl, sh, sl, w = _dft_constants()

    CB = C // 2
    const_specs = [
        pl.BlockSpec((S, FPAD), lambda i: (0, 0)) for _ in range(4)
    ]
    freq = pl.pallas_call(
        _fwd_dft_body,
        grid=(C // CB,),
        in_specs=const_specs + [
            pl.BlockSpec((S, CB), lambda i: (0, i)),
            pl.BlockSpec((S, CB), lambda i: (0, i)),
        ],
        out_specs=tuple(
            pl.BlockSpec((FPAD, CB), lambda i: (0, i)) for _ in range(4)
        ),
        out_shape=tuple(
            jax.ShapeDtypeStruct((FPAD, C), jnp.float32) for _ in range(4)
        ),
        compiler_params=pltpu.CompilerParams(
            vmem_limit_bytes=63 * 1024 * 1024,
        ),
    )(ch, cl, sh, sl, q2, k2)

    corr = pl.pallas_call(
        _inv_dft_body,
        grid=(C // CB,),
        in_specs=const_specs + [pl.BlockSpec((FPAD, 1), lambda i: (0, 0))] + [
            pl.BlockSpec((FPAD, CB), lambda i: (0, i)) for _ in range(4)
        ],
        out_specs=pl.BlockSpec((S, CB), lambda i: (0, i)),
        out_shape=jax.ShapeDtypeStruct((S, C), jnp.float32),
        compiler_params=pltpu.CompilerParams(
            vmem_limit_bytes=63 * 1024 * 1024,
        ),
    )(ch, cl, sh, sl, w, *freq)

    a, gidx = pl.pallas_call(
        _sort_softmax_body,
        grid=(C // SORT_CB,),
        in_specs=[pl.BlockSpec((S, SORT_CB), lambda i: (0, i))],
        out_specs=(
            pl.BlockSpec((S, SORT_CB), lambda i: (0, i)),
            pl.BlockSpec((S, SORT_CB), lambda i: (0, i)),
        ),
        out_shape=(
            jax.ShapeDtypeStruct((S, C), jnp.float32),
            jax.ShapeDtypeStruct((S, C), jnp.int32),
        ),
        scratch_shapes=[
            pltpu.VMEM((S, SORT_CB), jnp.float32),
            pltpu.VMEM((S, SORT_CB), jnp.int32),
        ],
        compiler_params=pltpu.CompilerParams(
            vmem_limit_bytes=63 * 1024 * 1024,
        ),
    )(corr)

    w_rows = jnp.swapaxes(a.reshape(S, H, D), 0, 1).reshape(ROWS, D)
    i_rows = jnp.swapaxes(gidx.reshape(S, H, D), 0, 1).reshape(ROWS, D)
    v_flat = values[0].reshape(ROWS, D)

    out = _sc_gather(v_flat, w_rows, i_rows)
    return out.reshape(1, H, S, D).astype(queries.dtype)


# speedup vs baseline: 10.3938x; 1.1595x over previous
import functools
import math

import jax
import jax.numpy as jnp
from jax import lax
from jax.experimental import pallas as pl
from jax.experimental.pallas import tpu as pltpu
from jax.experimental.pallas import tpu_sc as plsc

S = 2048
H = 12
D = 64
C = H * D
F = S // 2 + 1
FPAD = 1032
NC = 2
NS = 16
NW = NC * NS
ROWS = H * S
RPW = ROWS // NW
RB = 64


_LOSCALE = 256.0


def _bsplit(x):
    hi = x.astype(jnp.bfloat16)
    lo = ((x - hi.astype(jnp.float32)) * _LOSCALE).astype(jnp.bfloat16)
    return hi, lo


def _np_bsplit(x64):
    import numpy as np
    xf = np.asarray(x64, np.float32)
    hi = xf.astype(jnp.bfloat16)
    lo = ((xf - hi.astype(np.float32)) * np.float32(_LOSCALE)).astype(jnp.bfloat16)
    return hi, lo


def _dft_constants():
    import numpy as np
    sv = np.arange(S, dtype=np.int64)[:, None]
    fv = np.arange(FPAD, dtype=np.int64)[None, :]
    m = (sv * fv) % S
    ang = m.astype(np.float64) * (2.0 * math.pi / S)
    valid = (fv < F).astype(np.float64)
    cm = np.cos(ang) * valid
    sm = np.sin(ang) * valid
    w = np.where(fv == 0, 1.0 / S, 2.0 / S)
    w = (np.where(fv == F - 1, 1.0 / S, w) * valid).astype(np.float32)
    ch, cl = _np_bsplit(cm)
    sh, sl = _np_bsplit(sm)
    return (jnp.asarray(ch), jnp.asarray(cl), jnp.asarray(sh),
            jnp.asarray(sl), jnp.asarray(w.reshape(FPAD, 1)))


_DNT = (((0,), (0,)), ((), ()))
_DNN = (((1,), (0,)), ((), ()))


def _dot3(mh, ml, x, dn):
    xh, xl = _bsplit(x)
    d = lambda a, b: lax.dot_general(a, b, dn,
                                     preferred_element_type=jnp.float32)
    return d(mh, xh) + (d(mh, xl) + d(ml, xh)) * jnp.float32(1.0 / _LOSCALE)


def _fwd_dft_body(ch, cl, sh, sl, q_ref, k_ref, qr_ref, qi_ref, kr_ref, ki_ref):
    q = q_ref[...]
    k = k_ref[...]
    chv, clv, shv, slv = ch[...], cl[...], sh[...], sl[...]
    qr_ref[...] = _dot3(chv, clv, q, _DNT)
    qi_ref[...] = -_dot3(shv, slv, q, _DNT)
    kr_ref[...] = _dot3(chv, clv, k, _DNT)
    ki_ref[...] = -_dot3(shv, slv, k, _DNT)


def _inv_dft_body(ch, cl, sh, sl, w, qr_ref, qi_ref, kr_ref, ki_ref, corr_ref):
    qr, qi = qr_ref[...], qi_ref[...]
    kr, ki = kr_ref[...], ki_ref[...]
    wv = w[...]
    pr = (qr * kr + qi * ki) * wv
    pi = (qi * kr - qr * ki) * wv
    corr_ref[...] = (
        _dot3(ch[...], cl[...], pr, _DNN) - _dot3(sh[...], sl[...], pi, _DNN)
    )


SORT_CB = 128


def _sort_softmax_body(corr_ref, a_ref, gidx_ref, key_ref, idx_ref):
    cb = SORT_CB
    key_ref[...] = corr_ref[...]
    t2 = lax.broadcasted_iota(jnp.int32, (S, cb), 0)
    idx_ref[...] = t2

    def stage(d, ksz):
        key = key_ref[...]
        idx = idx_ref[...]
        tbit = (t2 & d) != 0
        desc = (t2 & ksz) == 0
        k_up = pltpu.roll(key, S - d, axis=0)
        k_dn = pltpu.roll(key, d, axis=0)
        i_up = pltpu.roll(idx, S - d, axis=0)
        i_dn = pltpu.roll(idx, d, axis=0)
        k_p = jnp.where(tbit, k_dn, k_up)
        i_p = jnp.where(tbit, i_dn, i_up)
        swap = jnp.equal(key < k_p, tbit ^ desc)
        key_ref[...] = jnp.where(swap, k_p, key)
        idx_ref[...] = jnp.where(swap, i_p, idx)

    def outer(k, carry):
        ksz = jnp.int32(1) << k

        def inner(j, c2):
            stage(jnp.int32(1) << (k - 1 - j), ksz)
            return c2

        lax.fori_loop(0, k, inner, 0)
        return carry

    lax.fori_loop(1, 12, outer, 0)
    key = key_ref[...]
    e = jnp.exp(key - key[0:1, :])
    a_ref[...] = e / jnp.sum(e, axis=0, keepdims=True)
    chan = pl.program_id(0) * cb + lax.broadcasted_iota(jnp.int32, (S, cb), 1)
    gidx_ref[...] = idx_ref[...] + (chan // D) * S


def _sc_gather_body(v_hbm, w_hbm, i_hbm, out_hbm,
                    idxb, wb, rows0, rows1, outb, sem0, sem1):
    wid = lax.axis_index("s") * NC + lax.axis_index("c")
    base = wid * RPW

    def compute(rows, rr):
        wvecs = [wb[rr, pl.ds(g * 16, 16)] for g in range(4)]
        accs = [jnp.zeros((16,), jnp.float32) for _ in range(4)]
        for j in range(D):
            wj = wvecs[j // 16][j % 16]
            for c in range(4):
                accs[c] = accs[c] + wj * rows[j, pl.ds(c * 16, 16)]
        for c in range(4):
            outb[rr, pl.ds(c * 16, 16)] = accs[c]

    def blk(bi, carry):
        r0 = base + bi * RB
        pltpu.sync_copy(i_hbm.at[pl.ds(r0, RB)], idxb)
        pltpu.sync_copy(w_hbm.at[pl.ds(r0, RB)], wb)
        pltpu.async_copy(v_hbm.at[idxb.at[0]], rows0, sem0)

        def pair(pp, c2):
            rr = pp * 2
            pltpu.async_copy(v_hbm.at[idxb.at[rr + 1]], rows1, sem1)
            pltpu.make_async_copy(v_hbm.at[idxb.at[rr]], rows0, sem0).wait()
            compute(rows0, rr)

            @pl.when(rr + 2 < RB)
            def _prefetch():
                pltpu.async_copy(v_hbm.at[idxb.at[rr + 2]], rows0, sem0)

            pltpu.make_async_copy(v_hbm.at[idxb.at[rr + 1]], rows1, sem1).wait()
            compute(rows1, rr + 1)
            return c2

        lax.fori_loop(0, RB // 2, pair, 0)
        pltpu.sync_copy(outb, out_hbm.at[pl.ds(r0, RB)])
        return carry

    lax.fori_loop(0, RPW // RB, blk, 0)


def _sc_gather(v_flat, w_rows, i_rows):
    mesh = plsc.VectorSubcoreMesh(core_axis_name="c", subcore_axis_name="s")
    run = pl.kernel(
        _sc_gather_body,
        out_type=jax.ShapeDtypeStruct((ROWS, D), jnp.float32),
        mesh=mesh,
        compiler_params=pltpu.CompilerParams(use_tc_tiling_on_sc=False),
        scratch_types=[
            pltpu.VMEM((RB, D), jnp.int32),
            pltpu.VMEM((RB, D), jnp.float32),
            pltpu.VMEM((D, D), jnp.float32),
            pltpu.VMEM((D, D), jnp.float32),
            pltpu.VMEM((RB, D), jnp.float32),
            pltpu.SemaphoreType.DMA,
            pltpu.SemaphoreType.DMA,
        ],
    )
    return run(v_flat, w_rows, i_rows)


def kernel(queries, keys, values):
    q2 = jnp.moveaxis(queries[0], 0, 1).reshape(S, C)
    k2 = jnp.moveaxis(keys[0], 0, 1).reshape(S, C)
    ch, cl, sh, sl, w = _dft_constants()

    CB = C // 2
    const_specs = [
        pl.BlockSpec((S, FPAD), lambda i: (0, 0)) for _ in range(4)
    ]
    freq = pl.pallas_call(
        _fwd_dft_body,
        grid=(C // CB,),
        in_specs=const_specs + [
            pl.BlockSpec((S, CB), lambda i: (0, i)),
            pl.BlockSpec((S, CB), lambda i: (0, i)),
        ],
        out_specs=tuple(
            pl.BlockSpec((FPAD, CB), lambda i: (0, i)) for _ in range(4)
        ),
        out_shape=tuple(
            jax.ShapeDtypeStruct((FPAD, C), jnp.float32) for _ in range(4)
        ),
        compiler_params=pltpu.CompilerParams(
            vmem_limit_bytes=63 * 1024 * 1024,
        ),
    )(ch, cl, sh, sl, q2, k2)

    corr = pl.pallas_call(
        _inv_dft_body,
        grid=(C // CB,),
        in_specs=const_specs + [pl.BlockSpec((FPAD, 1), lambda i: (0, 0))] + [
            pl.BlockSpec((FPAD, CB), lambda i: (0, i)) for _ in range(4)
        ],
        out_specs=pl.BlockSpec((S, CB), lambda i: (0, i)),
        out_shape=jax.ShapeDtypeStruct((S, C), jnp.float32),
        compiler_params=pltpu.CompilerParams(
            vmem_limit_bytes=63 * 1024 * 1024,
        ),
    )(ch, cl, sh, sl, w, *freq)

    a, gidx = pl.pallas_call(
        _sort_softmax_body,
        grid=(C // SORT_CB,),
        in_specs=[pl.BlockSpec((S, SORT_CB), lambda i: (0, i))],
        out_specs=(
            pl.BlockSpec((S, SORT_CB), lambda i: (0, i)),
            pl.BlockSpec((S, SORT_CB), lambda i: (0, i)),
        ),
        out_shape=(
            jax.ShapeDtypeStruct((S, C), jnp.float32),
            jax.ShapeDtypeStruct((S, C), jnp.int32),
        ),
        scratch_shapes=[
            pltpu.VMEM((S, SORT_CB), jnp.float32),
            pltpu.VMEM((S, SORT_CB), jnp.int32),
        ],
        compiler_params=pltpu.CompilerParams(
            vmem_limit_bytes=63 * 1024 * 1024,
        ),
    )(corr)

    w_rows = jnp.swapaxes(a.reshape(S, H, D), 0, 1).reshape(ROWS, D)
    i_rows = jnp.swapaxes(gidx.reshape(S, H, D), 0, 1).reshape(ROWS, D)
    v_flat = values[0].reshape(ROWS, D)

    out = _sc_gather(v_flat, w_rows, i_rows)
    return out.reshape(1, H, S, D).astype(queries.dtype)


# speedup vs baseline: 10.4087x; 1.0014x over previous
import functools
import math

import jax
import jax.numpy as jnp
from jax import lax
from jax.experimental import pallas as pl
from jax.experimental.pallas import tpu as pltpu
from jax.experimental.pallas import tpu_sc as plsc

S = 2048
H = 12
D = 64
C = H * D
F = S // 2 + 1
FPAD = 1032
NC = 2
NS = 16
NW = NC * NS
ROWS = H * S
RPW = ROWS // NW
RB = 64


_LOSCALE = 256.0


def _bsplit(x):
    hi = x.astype(jnp.bfloat16)
    lo = ((x - hi.astype(jnp.float32)) * _LOSCALE).astype(jnp.bfloat16)
    return hi, lo


def _np_bsplit(x64):
    import numpy as np
    xf = np.asarray(x64, np.float32)
    hi = xf.astype(jnp.bfloat16)
    lo = ((xf - hi.astype(np.float32)) * np.float32(_LOSCALE)).astype(jnp.bfloat16)
    return hi, lo


def _dft_constants():
    import numpy as np
    sv = np.arange(S, dtype=np.int64)[:, None]
    fv = np.arange(FPAD, dtype=np.int64)[None, :]
    m = (sv * fv) % S
    ang = m.astype(np.float64) * (2.0 * math.pi / S)
    valid = (fv < F).astype(np.float64)
    cm = np.cos(ang) * valid
    sm = np.sin(ang) * valid
    w = np.where(fv == 0, 1.0 / S, 2.0 / S)
    w = (np.where(fv == F - 1, 1.0 / S, w) * valid).astype(np.float32)
    ch, cl = _np_bsplit(cm)
    sh, sl = _np_bsplit(sm)
    return (jnp.asarray(ch), jnp.asarray(cl), jnp.asarray(sh),
            jnp.asarray(sl), jnp.asarray(w.reshape(FPAD, 1)))


_DNT = (((0,), (0,)), ((), ()))
_DNN = (((1,), (0,)), ((), ()))


def _dot3(mh, ml, x, dn):
    xh, xl = _bsplit(x)
    d = lambda a, b: lax.dot_general(a, b, dn,
                                     preferred_element_type=jnp.float32)
    return d(mh, xh) + (d(mh, xl) + d(ml, xh)) * jnp.float32(1.0 / _LOSCALE)


def _fwd_dft_body(ch, cl, sh, sl, q_ref, k_ref, qr_ref, qi_ref, kr_ref, ki_ref):
    q = q_ref[...]
    k = k_ref[...]
    chv, clv, shv, slv = ch[...], cl[...], sh[...], sl[...]
    qr_ref[...] = _dot3(chv, clv, q, _DNT)
    qi_ref[...] = -_dot3(shv, slv, q, _DNT)
    kr_ref[...] = _dot3(chv, clv, k, _DNT)
    ki_ref[...] = -_dot3(shv, slv, k, _DNT)


def _inv_dft_body(ch, cl, sh, sl, w, qr_ref, qi_ref, kr_ref, ki_ref, corr_ref):
    qr, qi = qr_ref[...], qi_ref[...]
    kr, ki = kr_ref[...], ki_ref[...]
    wv = w[...]
    pr = (qr * kr + qi * ki) * wv
    pi = (qi * kr - qr * ki) * wv
    corr_ref[...] = (
        _dot3(ch[...], cl[...], pr, _DNN) - _dot3(sh[...], sl[...], pi, _DNN)
    )


SORT_CB = 384


def _sort_softmax_body(corr_ref, a_ref, gidx_ref, key_ref, idx_ref):
    cb = SORT_CB
    key_ref[...] = corr_ref[...]
    t2 = lax.broadcasted_iota(jnp.int32, (S, cb), 0)
    idx_ref[...] = t2

    def stage(d, ksz):
        key = key_ref[...]
        idx = idx_ref[...]
        tbit = (t2 & d) != 0
        desc = (t2 & ksz) == 0
        k_up = pltpu.roll(key, S - d, axis=0)
        k_dn = pltpu.roll(key, d, axis=0)
        i_up = pltpu.roll(idx, S - d, axis=0)
        i_dn = pltpu.roll(idx, d, axis=0)
        k_p = jnp.where(tbit, k_dn, k_up)
        i_p = jnp.where(tbit, i_dn, i_up)
        swap = jnp.equal(key < k_p, tbit ^ desc)
        key_ref[...] = jnp.where(swap, k_p, key)
        idx_ref[...] = jnp.where(swap, i_p, idx)

    def outer(k, carry):
        ksz = jnp.int32(1) << k

        def inner(j, c2):
            stage(jnp.int32(1) << (k - 1 - j), ksz)
            return c2

        lax.fori_loop(0, k, inner, 0)
        return carry

    lax.fori_loop(1, 12, outer, 0)
    key = key_ref[...]
    e = jnp.exp(key - key[0:1, :])
    a_ref[...] = e / jnp.sum(e, axis=0, keepdims=True)
    chan = pl.program_id(0) * cb + lax.broadcasted_iota(jnp.int32, (S, cb), 1)
    gidx_ref[...] = idx_ref[...] + (chan // D) * S


def _sc_gather_body(v_hbm, w_hbm, i_hbm, out_hbm,
                    idxb, wb, rows0, rows1, outb, sem0, sem1):
    wid = lax.axis_index("s") * NC + lax.axis_index("c")
    base = wid * RPW

    def compute(rows, rr):
        wvecs = [wb[rr, pl.ds(g * 16, 16)] for g in range(4)]
        accs = [jnp.zeros((16,), jnp.float32) for _ in range(4)]
        for j in range(D):
            wj = wvecs[j // 16][j % 16]
            for c in range(4):
                accs[c] = accs[c] + wj * rows[j, pl.ds(c * 16, 16)]
        for c in range(4):
            outb[rr, pl.ds(c * 16, 16)] = accs[c]

    def blk(bi, carry):
        r0 = base + bi * RB
        pltpu.sync_copy(i_hbm.at[pl.ds(r0, RB)], idxb)
        pltpu.sync_copy(w_hbm.at[pl.ds(r0, RB)], wb)
        pltpu.async_copy(v_hbm.at[idxb.at[0]], rows0, sem0)

        def pair(pp, c2):
            rr = pp * 2
            pltpu.async_copy(v_hbm.at[idxb.at[rr + 1]], rows1, sem1)
            pltpu.make_async_copy(v_hbm.at[idxb.at[rr]], rows0, sem0).wait()
            compute(rows0, rr)

            @pl.when(rr + 2 < RB)
            def _prefetch():
                pltpu.async_copy(v_hbm.at[idxb.at[rr + 2]], rows0, sem0)

            pltpu.make_async_copy(v_hbm.at[idxb.at[rr + 1]], rows1, sem1).wait()
            compute(rows1, rr + 1)
            return c2

        lax.fori_loop(0, RB // 2, pair, 0)
        pltpu.sync_copy(outb, out_hbm.at[pl.ds(r0, RB)])
        return carry

    lax.fori_loop(0, RPW // RB, blk, 0)


def _sc_gather(v_flat, w_rows, i_rows):
    mesh = plsc.VectorSubcoreMesh(core_axis_name="c", subcore_axis_name="s")
    run = pl.kernel(
        _sc_gather_body,
        out_type=jax.ShapeDtypeStruct((ROWS, D), jnp.float32),
        mesh=mesh,
        compiler_params=pltpu.CompilerParams(use_tc_tiling_on_sc=False),
        scratch_types=[
            pltpu.VMEM((RB, D), jnp.int32),
            pltpu.VMEM((RB, D), jnp.float32),
            pltpu.VMEM((D, D), jnp.float32),
            pltpu.VMEM((D, D), jnp.float32),
            pltpu.VMEM((RB, D), jnp.float32),
            pltpu.SemaphoreType.DMA,
            pltpu.SemaphoreType.DMA,
        ],
    )
    return run(v_flat, w_rows, i_rows)


def kernel(queries, keys, values):
    q2 = jnp.moveaxis(queries[0], 0, 1).reshape(S, C)
    k2 = jnp.moveaxis(keys[0], 0, 1).reshape(S, C)
    ch, cl, sh, sl, w = _dft_constants()

    CB = C // 2
    const_specs = [
        pl.BlockSpec((S, FPAD), lambda i: (0, 0)) for _ in range(4)
    ]
    freq = pl.pallas_call(
        _fwd_dft_body,
        grid=(C // CB,),
        in_specs=const_specs + [
            pl.BlockSpec((S, CB), lambda i: (0, i)),
            pl.BlockSpec((S, CB), lambda i: (0, i)),
        ],
        out_specs=tuple(
            pl.BlockSpec((FPAD, CB), lambda i: (0, i)) for _ in range(4)
        ),
        out_shape=tuple(
            jax.ShapeDtypeStruct((FPAD, C), jnp.float32) for _ in range(4)
        ),
        compiler_params=pltpu.CompilerParams(
            vmem_limit_bytes=63 * 1024 * 1024,
        ),
    )(ch, cl, sh, sl, q2, k2)

    corr = pl.pallas_call(
        _inv_dft_body,
        grid=(C // CB,),
        in_specs=const_specs + [pl.BlockSpec((FPAD, 1), lambda i: (0, 0))] + [
            pl.BlockSpec((FPAD, CB), lambda i: (0, i)) for _ in range(4)
        ],
        out_specs=pl.BlockSpec((S, CB), lambda i: (0, i)),
        out_shape=jax.ShapeDtypeStruct((S, C), jnp.float32),
        compiler_params=pltpu.CompilerParams(
            vmem_limit_bytes=63 * 1024 * 1024,
        ),
    )(ch, cl, sh, sl, w, *freq)

    a, gidx = pl.pallas_call(
        _sort_softmax_body,
        grid=(C // SORT_CB,),
        in_specs=[pl.BlockSpec((S, SORT_CB), lambda i: (0, i))],
        out_specs=(
            pl.BlockSpec((S, SORT_CB), lambda i: (0, i)),
            pl.BlockSpec((S, SORT_CB), lambda i: (0, i)),
        ),
        out_shape=(
            jax.ShapeDtypeStruct((S, C), jnp.float32),
            jax.ShapeDtypeStruct((S, C), jnp.int32),
        ),
        scratch_shapes=[
            pltpu.VMEM((S, SORT_CB), jnp.float32),
            pltpu.VMEM((S, SORT_CB), jnp.int32),
        ],
        compiler_params=pltpu.CompilerParams(
            vmem_limit_bytes=63 * 1024 * 1024,
        ),
    )(corr)

    w_rows = jnp.swapaxes(a.reshape(S, H, D), 0, 1).reshape(ROWS, D)
    i_rows = jnp.swapaxes(gidx.reshape(S, H, D), 0, 1).reshape(ROWS, D)
    v_flat = values[0].reshape(ROWS, D)

    out = _sc_gather(v_flat, w_rows, i_rows)
    return out.reshape(1, H, S, D).astype(queries.dtype)
